# baseline (speedup 1.0000x reference)
import jax
import jax.numpy as jnp
from jax import lax
from jax.experimental import pallas as pl
from jax.experimental.pallas import tpu as pltpu
from jax.experimental.pallas import tpu_sc as plsc

TOP_K = 16
GB = 32
LT = 512
NC, NS = 2, 16
LANES = 16


def _sim_kernel(feat_ref, keys_ref, sim_ref):
    f = feat_ref[...]
    k = keys_ref[:, 0, :]
    dots = lax.dot_general(
        f, k, (((1,), (1,)), ((), ())),
        preferred_element_type=jnp.float32)
    qn = jnp.sqrt(jnp.sum(f * f, axis=1, keepdims=True))
    kn = jnp.sqrt(jnp.sum(k * k, axis=1, keepdims=True))
    sim_ref[...] = dots / jnp.maximum(qn * kn.T, 1e-8)


def _make_topk_sc(B, M):
    nchunk = M // LANES
    rows_per = B // (NC * NS)

    def _topk_weights(sim_hbm, w_hbm, row_v, wrow_v):
        wid = lax.axis_index("c") * NS + lax.axis_index("s")
        iota = lax.broadcasted_iota(jnp.int32, (LANES,), 0)
        zero = jnp.zeros((LANES,), jnp.float32)
        neg_inf = jnp.float32(-jnp.inf)

        @pl.loop(0, rows_per)
        def _row(r):
            row = wid * rows_per + r
            pltpu.sync_copy(sim_hbm.at[row], row_v)
            for c in range(nchunk):
                wrow_v[pl.ds(c * LANES, LANES)] = zero

            total = jnp.float32(0.0)
            for _ in range(TOP_K):
                m = row_v[pl.ds(0, LANES)]
                for c in range(1, nchunk):
                    m = jnp.maximum(m, row_v[pl.ds(c * LANES, LANES)])
                mval = jnp.max(m)
                best = jnp.int32(M)
                for c in range(nchunk):
                    v = row_v[pl.ds(c * LANES, LANES)]
                    cand = jnp.where(v == mval, iota + c * LANES, M)
                    best = jnp.minimum(best, jnp.min(cand))
                wsel = jnp.maximum(mval, 0.0)
                for c in range(nchunk):
                    hit = (iota + c * LANES) == best
                    v = row_v[pl.ds(c * LANES, LANES)]
                    row_v[pl.ds(c * LANES, LANES)] = jnp.where(hit, neg_inf, v)
                    wv = wrow_v[pl.ds(c * LANES, LANES)]
                    wrow_v[pl.ds(c * LANES, LANES)] = jnp.where(hit, wsel, wv)
                total = total + wsel

            inv = jnp.ones((LANES,), jnp.float32) / (total + zero)
            for c in range(nchunk):
                wrow_v[pl.ds(c * LANES, LANES)] = (
                    wrow_v[pl.ds(c * LANES, LANES)] * inv)
            pltpu.sync_copy(wrow_v, w_hbm.at[row])

    return _topk_weights


def _mix_kernel(w_ref, mem_ref, out_ref):
    w = w_ref[...]
    for g in range(GB):
        out_ref[:, g, :] = lax.dot_general(
            w, mem_ref[:, g, :], (((1,), (0,)), ((), ())),
            preferred_element_type=jnp.float32)


@jax.jit
def kernel(features_, keys, memory):
    B, D = features_.shape
    M = keys.shape[0]
    G, L = memory.shape[1], memory.shape[2]

    sim = pl.pallas_call(
        _sim_kernel,
        out_shape=jax.ShapeDtypeStruct((B, M), jnp.float32),
    )(features_, keys)

    w = pl.kernel(
        _make_topk_sc(B, M),
        out_type=jax.ShapeDtypeStruct((B, M), jnp.float32),
        mesh=plsc.VectorSubcoreMesh(
            core_axis_name="c", subcore_axis_name="s",
            num_cores=NC, num_subcores=NS),
        scratch_types=[
            pltpu.VMEM((M,), jnp.float32),
            pltpu.VMEM((M,), jnp.float32),
        ],
        compiler_params=pltpu.CompilerParams(needs_layout_passes=False),
    )(sim)

    ctx = pl.pallas_call(
        _mix_kernel,
        grid=(G // GB, L // LT),
        in_specs=[
            pl.BlockSpec((B, M), lambda i, j: (0, 0)),
            pl.BlockSpec((M, GB, LT), lambda i, j: (0, i, j)),
        ],
        out_specs=pl.BlockSpec((B, GB, LT), lambda i, j: (0, i, j)),
        out_shape=jax.ShapeDtypeStruct((B, G, L), jnp.float32),
    )(w, memory)
    return ctx


# speedup vs baseline: 1.0255x; 1.0255x over previous
import jax
import jax.numpy as jnp
from jax import lax
from jax.experimental import pallas as pl
from jax.experimental.pallas import tpu as pltpu
from jax.experimental.pallas import tpu_sc as plsc

TOP_K = 16
GB = 32
LT = 512
NC, NS = 2, 16
LANES = 16


def _sim_kernel(feat_ref, keys_ref, sim_ref):
    f = feat_ref[...]
    k = keys_ref[:, 0, :]
    dots = lax.dot_general(
        f, k, (((1,), (1,)), ((), ())),
        preferred_element_type=jnp.float32)
    qn = jnp.sqrt(jnp.sum(f * f, axis=1, keepdims=True))
    kn = jnp.sqrt(jnp.sum(k * k, axis=1, keepdims=True))
    sim_ref[...] = dots / jnp.maximum(qn * kn.T, 1e-8)


def _make_topk_sc(B, M):
    nchunk = M // LANES
    rows_per = B // (NC * NS)

    def _topk_weights(sim_hbm, w_hbm, row_v, wrow_v):
        wid = lax.axis_index("c") * NS + lax.axis_index("s")
        iota = lax.broadcasted_iota(jnp.int32, (LANES,), 0)
        giota = [iota + c * LANES for c in range(nchunk)]
        zero = jnp.zeros((LANES,), jnp.float32)
        neg_inf = jnp.float32(-jnp.inf)

        @pl.loop(0, rows_per)
        def _row(r):
            row = wid * rows_per + r
            pltpu.sync_copy(sim_hbm.at[row], row_v)
            work = [row_v[pl.ds(c * LANES, LANES)] for c in range(nchunk)]
            sel = [iota < 0 for c in range(nchunk)]

            total = jnp.float32(0.0)
            for _ in range(TOP_K):
                m = work[0]
                for c in range(1, nchunk):
                    m = jnp.maximum(m, work[c])
                mval = jnp.max(m)
                best = jnp.int32(M)
                for c in range(nchunk):
                    cand = jnp.where(work[c] == mval, giota[c], M)
                    best = jnp.minimum(best, jnp.min(cand))
                for c in range(nchunk):
                    hit = giota[c] == best
                    work[c] = jnp.where(hit, neg_inf, work[c])
                    sel[c] = jnp.logical_or(sel[c], hit)
                total = total + jnp.maximum(mval, 0.0)

            inv = jnp.ones((LANES,), jnp.float32) / (total + zero)
            for c in range(nchunk):
                orig = row_v[pl.ds(c * LANES, LANES)]
                w = jnp.where(sel[c], jnp.maximum(orig, 0.0), 0.0)
                wrow_v[pl.ds(c * LANES, LANES)] = w * inv
            pltpu.sync_copy(wrow_v, w_hbm.at[row])

    return _topk_weights


def _mix_kernel(w_ref, mem_ref, out_ref):
    w = w_ref[...]
    for g in range(GB):
        out_ref[:, g, :] = lax.dot_general(
            w, mem_ref[:, g, :], (((1,), (0,)), ((), ())),
            preferred_element_type=jnp.float32)


@jax.jit
def kernel(features_, keys, memory):
    B, D = features_.shape
    M = keys.shape[0]
    G, L = memory.shape[1], memory.shape[2]

    sim = pl.pallas_call(
        _sim_kernel,
        out_shape=jax.ShapeDtypeStruct((B, M), jnp.float32),
    )(features_, keys)

    w = pl.kernel(
        _make_topk_sc(B, M),
        out_type=jax.ShapeDtypeStruct((B, M), jnp.float32),
        mesh=plsc.VectorSubcoreMesh(
            core_axis_name="c", subcore_axis_name="s",
            num_cores=NC, num_subcores=NS),
        scratch_types=[
            pltpu.VMEM((M,), jnp.float32),
            pltpu.VMEM((M,), jnp.float32),
        ],
        compiler_params=pltpu.CompilerParams(needs_layout_passes=False),
    )(sim)

    ctx = pl.pallas_call(
        _mix_kernel,
        grid=(G // GB, L // LT),
        in_specs=[
            pl.BlockSpec((B, M), lambda i, j: (0, 0)),
            pl.BlockSpec((M, GB, LT), lambda i, j: (0, i, j)),
        ],
        out_specs=pl.BlockSpec((B, GB, LT), lambda i, j: (0, i, j)),
        out_shape=jax.ShapeDtypeStruct((B, G, L), jnp.float32),
    )(w, memory)
    return ctx


# speedup vs baseline: 1.0313x; 1.0057x over previous
import jax
import jax.numpy as jnp
from jax import lax
from jax.experimental import pallas as pl
from jax.experimental.pallas import tpu as pltpu
from jax.experimental.pallas import tpu_sc as plsc

TOP_K = 16
GB = 32
LT = 512
NC, NS = 2, 16
LANES = 16


def _sim_kernel(feat_ref, keys_ref, sim_ref):
    f = feat_ref[...]
    k = keys_ref[:, 0, :]
    dots = lax.dot_general(
        f, k, (((1,), (1,)), ((), ())),
        preferred_element_type=jnp.float32)
    qn = jnp.sqrt(jnp.sum(f * f, axis=1, keepdims=True))
    kn = jnp.sqrt(jnp.sum(k * k, axis=1, keepdims=True))
    sim_ref[...] = dots / jnp.maximum(qn * kn.T, 1e-8)


def _make_topk_sc(B, M):
    nchunk = M // LANES
    rows_per = B // (NC * NS)

    def _topk_weights(sim_hbm, w_hbm, rows_v, wrows_v):
        wid = lax.axis_index("c") * NS + lax.axis_index("s")
        base = wid * rows_per
        iota = lax.broadcasted_iota(jnp.int32, (LANES,), 0)
        giota = [iota + c * LANES for c in range(nchunk)]
        zero = jnp.zeros((LANES,), jnp.float32)
        neg_inf = jnp.float32(-jnp.inf)

        pltpu.sync_copy(sim_hbm.at[pl.ds(base, rows_per)], rows_v)

        @pl.loop(0, rows_per)
        def _row(r):
            work = [rows_v[r, pl.ds(c * LANES, LANES)] for c in range(nchunk)]
            sel = [iota < 0 for c in range(nchunk)]

            total = jnp.float32(0.0)
            for _ in range(TOP_K):
                m = work[0]
                for c in range(1, nchunk):
                    m = jnp.maximum(m, work[c])
                mval = jnp.max(m)
                best = jnp.int32(M)
                for c in range(nchunk):
                    cand = jnp.where(work[c] == mval, giota[c], M)
                    best = jnp.minimum(best, jnp.min(cand))
                for c in range(nchunk):
                    hit = giota[c] == best
                    work[c] = jnp.where(hit, neg_inf, work[c])
                    sel[c] = jnp.logical_or(sel[c], hit)
                total = total + jnp.maximum(mval, 0.0)

            inv = jnp.ones((LANES,), jnp.float32) / (total + zero)
            for c in range(nchunk):
                orig = rows_v[r, pl.ds(c * LANES, LANES)]
                w = jnp.where(sel[c], jnp.maximum(orig, 0.0), 0.0)
                wrows_v[r, pl.ds(c * LANES, LANES)] = w * inv

        pltpu.sync_copy(wrows_v, w_hbm.at[pl.ds(base, rows_per)])

    return _topk_weights


def _mix_kernel(w_ref, mem_ref, out_ref):
    w = w_ref[...]
    for g in range(GB):
        out_ref[:, g, :] = lax.dot_general(
            w, mem_ref[:, g, :], (((1,), (0,)), ((), ())),
            preferred_element_type=jnp.float32)


@jax.jit
def kernel(features_, keys, memory):
    B, D = features_.shape
    M = keys.shape[0]
    G, L = memory.shape[1], memory.shape[2]

    sim = pl.pallas_call(
        _sim_kernel,
        out_shape=jax.ShapeDtypeStruct((B, M), jnp.float32),
    )(features_, keys)

    w = pl.kernel(
        _make_topk_sc(B, M),
        out_type=jax.ShapeDtypeStruct((B, M), jnp.float32),
        mesh=plsc.VectorSubcoreMesh(
            core_axis_name="c", subcore_axis_name="s",
            num_cores=NC, num_subcores=NS),
        scratch_types=[
            pltpu.VMEM((B // (NC * NS), M), jnp.float32),
            pltpu.VMEM((B // (NC * NS), M), jnp.float32),
        ],
        compiler_params=pltpu.CompilerParams(needs_layout_passes=False),
    )(sim)

    ctx = pl.pallas_call(
        _mix_kernel,
        grid=(G // GB, L // LT),
        in_specs=[
            pl.BlockSpec((B, M), lambda i, j: (0, 0)),
            pl.BlockSpec((M, GB, LT), lambda i, j: (0, i, j)),
        ],
        out_specs=pl.BlockSpec((B, GB, LT), lambda i, j: (0, i, j)),
        out_shape=jax.ShapeDtypeStruct((B, G, L), jnp.float32),
    )(w, memory)
    return ctx
